# baseline (speedup 1.0000x reference)
import functools

import jax
import jax.numpy as jnp
from jax import lax
from jax.experimental import pallas as pl
from jax.experimental.pallas import tpu as pltpu
from jax.experimental.pallas import tpu_sc as plsc

_D = 128
_B = 512
_NW = 32
_BW = _B // _NW
_L = 16


def _sc_partials(table, rel, nfall):
    mesh = plsc.VectorSubcoreMesh(core_axis_name="c", subcore_axis_name="s")
    f32 = jnp.float32
    i32 = jnp.int32

    @functools.partial(
        pl.kernel,
        mesh=mesh,
        compiler_params=pltpu.CompilerParams(
            use_tc_tiling_on_sc=False, needs_layout_passes=False),
        out_type=jax.ShapeDtypeStruct((4 * _L * _NW,), f32),
        scratch_types=(
            pltpu.VMEM((15 * _BW,), i32),
            pltpu.VMEM((4 * _BW, _D), f32),
            pltpu.VMEM((6 * _BW, _D), f32),
            pltpu.VMEM((4 * _BW, _D), f32),
            pltpu.VMEM((_BW, _D), f32),
            pltpu.VMEM((_BW * (_L + 1),), f32),
            pltpu.VMEM((_BW * (_L + 1),), f32),
            pltpu.VMEM((4 * _L,), f32),
            pltpu.SemaphoreType.DMA,
            pltpu.SemaphoreType.DMA,
            pltpu.SemaphoreType.DMA,
            pltpu.SemaphoreType.DMA,
            pltpu.SemaphoreType.DMA,
        ),
    )
    def k(table_hbm, rel_hbm, nf_hbm, out_hbm,
          xi, v1, v2, v3, vr, accma, accmb, sall,
          sim, g1m, g2m, g3m, grm):
        wid = lax.axis_index("s") * 2 + lax.axis_index("c")

        pltpu.async_copy(
            nf_hbm.at[pl.ds(wid * 15 * _BW, 15 * _BW)], xi, sim).wait()

        cp1 = pltpu.async_copy(table_hbm.at[xi.at[pl.ds(0, 4 * _BW)]], v1, g1m)
        cp2 = pltpu.async_copy(
            table_hbm.at[xi.at[pl.ds(4 * _BW, 6 * _BW)]], v2, g2m)
        cp3 = pltpu.async_copy(
            table_hbm.at[xi.at[pl.ds(10 * _BW, 4 * _BW)]], v3, g3m)
        cpr = pltpu.async_copy(
            rel_hbm.at[xi.at[pl.ds(14 * _BW, _BW)]], vr, grm)

        lanes = lax.broadcasted_iota(i32, (_BW,), 0)

        cp1.wait()

        def body1(r, acc):
            def chunk1(j, a):
                lo = pl.ds(_L * j, _L)
                c1 = v1[r, lo]
                cr = v1[r + _BW, lo]
                d1 = v1[r + 2 * _BW, lo]
                dr = v1[r + 3 * _BW, lo]
                t = jnp.maximum(
                    jnp.abs(c1 - d1) + jnp.abs(cr) - jnp.abs(dr), 0.0)
                return a + t * t

            return lax.fori_loop(0, _D // _L, chunk1, acc)

        acc1 = plsc.parallel_loop(
            0, _BW, unroll=1, carry=jnp.zeros((_L,), f32))(body1)
        sall[pl.ds(0, _L)] = acc1

        cp2.wait()

        @plsc.parallel_loop(0, _BW, unroll=1)
        def body2(r):
            def chunk2(j, accs):
                aa, ab = accs
                lo = pl.ds(_L * j, _L)
                c1 = v2[r, lo]
                c2 = jnp.abs(v2[r + _BW, lo])
                d1 = v2[r + 2 * _BW, lo]
                d2 = jnp.abs(v2[r + 3 * _BW, lo])
                e1 = v2[r + 4 * _BW, lo]
                e2 = jnp.abs(v2[r + 5 * _BW, lo])
                start = jnp.maximum(c1 - c2, d1 - d2)
                end = jnp.minimum(c1 + c2, d1 + d2)
                diff = start - end
                cen = (start + end) * 0.5
                t1 = jnp.maximum(
                    jnp.abs(cen - e1) + jnp.abs(diff) * 0.5 - e2, 0.0)
                t2 = jnp.maximum(diff, 0.0)
                return (aa + t1 * t1, ab + t2 * t2)

            aa, ab = lax.fori_loop(
                0, _D // _L, chunk2,
                (jnp.zeros((_L,), f32), jnp.zeros((_L,), f32)))
            accma[pl.ds(r * (_L + 1), _L)] = aa
            accmb[pl.ds(r * (_L + 1), _L)] = ab

        cp3.wait()
        cpr.wait()

        def body3(r, acc):
            def chunk3(j, a):
                lo = pl.ds(_L * j, _L)
                c1 = v3[r, lo]
                cr = v3[r + _BW, lo]
                d1 = v3[r + 2 * _BW, lo]
                dr = v3[r + 3 * _BW, lo]
                rr = vr[r, lo]
                t = jnp.maximum(
                    jnp.abs(c1 + rr - d1) + jnp.abs(cr) - jnp.abs(dr), 0.0)
                return a + t * t

            return lax.fori_loop(0, _D // _L, chunk3, acc)

        acc3 = plsc.parallel_loop(
            0, _BW, unroll=1, carry=jnp.zeros((_L,), f32))(body3)
        sall[pl.ds(3 * _L, _L)] = acc3

        sa = jnp.zeros((_L,), f32)
        sb = jnp.zeros((_L,), f32)
        lanes17 = lanes * (_L + 1)
        for c in range(_L):
            sa = sa + plsc.load_gather(accma, [lanes17 + c])
            sb = sb + plsc.load_gather(accmb, [lanes17 + c])
        sall[pl.ds(_L, _L)] = sa
        sall[pl.ds(2 * _L, _L)] = sb

        pltpu.sync_copy(sall, out_hbm.at[pl.ds(wid * 4 * _L, 4 * _L)])

    return k(table, rel, nfall)


def _reduce_body(p_ref, o_ref):
    p = p_ref[...]

    def blk(k):
        return jnp.sum(p[:, k * _L:(k + 1) * _L]) \
            + jnp.sum(p[:, 64 + k * _L:64 + (k + 1) * _L])

    def blk_sqrt(k):
        return jnp.sum(jnp.sqrt(p[:, k * _L:(k + 1) * _L])) \
            + jnp.sum(jnp.sqrt(p[:, 64 + k * _L:64 + (k + 1) * _L]))

    inv = 1.0 / _B
    loss = (blk(0) + blk(1) + blk(2) + blk(3)) * inv \
        + 2.0 * (blk_sqrt(1) * inv) * (blk_sqrt(2) * inv)
    o_ref[...] = jnp.full((1, 1), loss, jnp.float32)


def kernel(class_emb, rel_emb, nf1, nf2, nf3):
    i32 = jnp.int32
    table = class_emb.reshape(2 * class_emb.shape[0], _D)
    cls_cols = jnp.concatenate(
        [nf1, nf2, nf3[:, 0:1], nf3[:, 2:3]], axis=1).astype(i32)
    pairs = jnp.stack(
        [2 * cls_cols, 2 * cls_cols + 1], axis=-1).reshape(_B, 14)
    x = jnp.concatenate([pairs, nf3[:, 1:2].astype(i32)], axis=1)
    nfall = x.reshape(_NW, _BW, 15).transpose(0, 2, 1).reshape(-1)
    partials = _sc_partials(table, rel_emb, nfall)
    out = pl.pallas_call(
        _reduce_body,
        out_shape=jax.ShapeDtypeStruct((1, 1), jnp.float32),
    )(partials.reshape(_L, 2 * 64))
    return out[0, 0]
